# baseline (speedup 1.0000x reference)
import functools

import jax
import jax.numpy as jnp
from jax import lax
from jax.experimental import pallas as pl
from jax.experimental.pallas import tpu as pltpu
from jax.experimental.pallas import tpu_sc as plsc

N = 10000
D = 128
E = 320000
COEFF = 0.1
NC = 2
NS = 16
C = 128
NCHUNK = 80
E2 = NC * NS * NCHUNK * C
NP = 10240
RPT = NP // NS


def _prepass_body(z_ref, nf_ref, zr_ref):
    zr_ref[...] = z_ref[...] * nf_ref[...]


def _prepass(z, nf):
    blk = 1000
    return pl.pallas_call(
        _prepass_body,
        grid=(N // blk,),
        in_specs=[
            pl.BlockSpec((blk, D), lambda i: (i, 0)),
            pl.BlockSpec((blk, 1), lambda i: (i, 0)),
        ],
        out_specs=pl.BlockSpec((blk, D), lambda i: (i, 0)),
        out_shape=jax.ShapeDtypeStruct((N, D), jnp.float32),
    )(z, nf)


def _postpass_body(z_ref, nf_ref, sa_ref, sb_ref, out_ref):
    s = sa_ref[...] + sb_ref[...]
    out_ref[...] = z_ref[...] - (2.0 * COEFF / N) * nf_ref[...] * s


def _postpass(z, nf, sa, sb):
    blk = 1000
    return pl.pallas_call(
        _postpass_body,
        grid=(N // blk,),
        in_specs=[
            pl.BlockSpec((blk, D), lambda i: (i, 0)),
            pl.BlockSpec((blk, 1), lambda i: (i, 0)),
            pl.BlockSpec((blk, D), lambda i: (i, 0)),
            pl.BlockSpec((blk, D), lambda i: (i, 0)),
        ],
        out_specs=pl.BlockSpec((blk, D), lambda i: (i, 0)),
        out_shape=jax.ShapeDtypeStruct((N, D), jnp.float32),
    )(z, nf, sa, sb)


def _diff(buf_a, buf_b):
    def body(i, _):
        for f16 in range(D // 16):
            sl = pl.ds(f16 * 16, 16)
            a = buf_a[i, sl]
            b = buf_b[i, sl]
            buf_a[i, sl] = a - b
            buf_b[i, sl] = b - a
        return 0

    lax.fori_loop(0, C, body, 0, unroll=2)


def _sc_body(zr, rows, cols, zeros,
             sa, sb,
             idx_r, idx_c, buf_a, buf_b, acc,
             isem, gsem, ssem):
    c = lax.axis_index("c")
    s = lax.axis_index("s")
    wid = c * NS + s
    base = wid * NCHUNK * C

    r0 = s * RPT
    pltpu.sync_copy(zeros.at[pl.ds(r0, RPT)], acc.at[pl.ds(r0, RPT)])
    plsc.subcore_barrier()

    @pl.loop(0, NCHUNK)
    def _(k):
        off = base + k * C
        i0 = pltpu.async_copy(rows.at[pl.ds(off, C)], idx_r, isem)
        i1 = pltpu.async_copy(cols.at[pl.ds(off, C)], idx_c, isem)
        i0.wait()
        i1.wait()
        ga = pltpu.async_copy(zr.at[idx_r], buf_a, gsem)
        gb = pltpu.async_copy(zr.at[idx_c], buf_b, gsem)
        ga.wait()
        gb.wait()
        _diff(buf_a, buf_b)
        sa_ = pltpu.async_copy(buf_a, acc.at[idx_r], ssem, add=True)
        sb_ = pltpu.async_copy(buf_b, acc.at[idx_c], ssem, add=True)
        sa_.wait()
        sb_.wait()

    plsc.subcore_barrier()

    @pl.when(c == 0)
    def _():
        pltpu.sync_copy(acc.at[pl.ds(r0, RPT)], sa.at[pl.ds(r0, RPT)])

    @pl.when(c == 1)
    def _():
        pltpu.sync_copy(acc.at[pl.ds(r0, RPT)], sb.at[pl.ds(r0, RPT)])


_sc_kernel = functools.partial(
    pl.kernel,
    out_type=[
        jax.ShapeDtypeStruct((NP, D), jnp.float32),
        jax.ShapeDtypeStruct((NP, D), jnp.float32),
    ],
    mesh=plsc.VectorSubcoreMesh(
        core_axis_name="c", subcore_axis_name="s",
        num_cores=NC, num_subcores=NS,
    ),
    scratch_types=[
        pltpu.VMEM((C,), jnp.int32),
        pltpu.VMEM((C,), jnp.int32),
        pltpu.VMEM((C, D), jnp.float32),
        pltpu.VMEM((C, D), jnp.float32),
        pltpu.VMEM_SHARED((NP, D), jnp.float32),
        pltpu.SemaphoreType.DMA,
        pltpu.SemaphoreType.DMA,
        pltpu.SemaphoreType.DMA,
    ],
)(_sc_body)


@jax.jit
def kernel(z, x, edge_index, norm_factor):
    del x
    zr = _prepass(z, norm_factor)
    pad = jnp.zeros((E2 - E,), jnp.int32)
    rows1 = jnp.concatenate([edge_index[0], pad])
    cols1 = jnp.concatenate([edge_index[1], pad])
    zeros = jnp.zeros((NP, D), jnp.float32)
    sa, sb = _sc_kernel(zr, rows1, cols1, zeros)
    return _postpass(z, norm_factor, sa, sb)


# speedup vs baseline: 2.3370x; 2.3370x over previous
import functools

import jax
import jax.numpy as jnp
from jax import lax
from jax.experimental import pallas as pl
from jax.experimental.pallas import tpu as pltpu
from jax.experimental.pallas import tpu_sc as plsc

N = 10000
D = 128
E = 320000
COEFF = 0.1
NC = 2
NS = 16
EPT = E // (NC * NS)
C = 80
NCHUNK = EPT // C
NP = 10240
RPT = NP // NS


def _prepass_body(z_ref, nf_ref, zr_ref):
    zr_ref[...] = z_ref[...] * nf_ref[...]


def _prepass(z, nf):
    blk = 1000
    return pl.pallas_call(
        _prepass_body,
        grid=(N // blk,),
        in_specs=[
            pl.BlockSpec((blk, D), lambda i: (i, 0)),
            pl.BlockSpec((blk, 1), lambda i: (i, 0)),
        ],
        out_specs=pl.BlockSpec((blk, D), lambda i: (i, 0)),
        out_shape=jax.ShapeDtypeStruct((N, D), jnp.float32),
    )(z, nf)


def _postpass_body(z_ref, nf_ref, sa_ref, sb_ref, out_ref):
    s = sa_ref[...] + sb_ref[...]
    out_ref[...] = z_ref[...] - (2.0 * COEFF / N) * nf_ref[...] * s


def _postpass(z, nf, sa, sb):
    blk = 1000
    return pl.pallas_call(
        _postpass_body,
        grid=(N // blk,),
        in_specs=[
            pl.BlockSpec((blk, D), lambda i: (i, 0)),
            pl.BlockSpec((blk, 1), lambda i: (i, 0)),
            pl.BlockSpec((blk, D), lambda i: (i, 0)),
            pl.BlockSpec((blk, D), lambda i: (i, 0)),
        ],
        out_specs=pl.BlockSpec((blk, D), lambda i: (i, 0)),
        out_shape=jax.ShapeDtypeStruct((N, D), jnp.float32),
    )(z, nf, sa, sb)


def _sc_body(zr, rows, cols, zeros,
             sa, sb,
             idx_r, idx_c, buf_a, buf_b, acc, sem):
    c = lax.axis_index("c")
    s = lax.axis_index("s")

    r0 = s * RPT
    pltpu.sync_copy(zeros.at[pl.ds(r0, RPT)], acc.at[pl.ds(r0, RPT)])
    plsc.subcore_barrier()

    base = (c * NS + s) * EPT

    def chunk(k, _):
        off = base + k * C
        pltpu.sync_copy(rows.at[pl.ds(off, C)], idx_r)
        pltpu.sync_copy(cols.at[pl.ds(off, C)], idx_c)
        pltpu.async_copy(zr.at[idx_r], buf_a, sem).wait()
        pltpu.async_copy(zr.at[idx_c], buf_b, sem).wait()

        def diff(i, _):
            for f16 in range(D // 16):
                a = buf_a[i, pl.ds(f16 * 16, 16)]
                b = buf_b[i, pl.ds(f16 * 16, 16)]
                buf_a[i, pl.ds(f16 * 16, 16)] = a - b
                buf_b[i, pl.ds(f16 * 16, 16)] = b - a
            return 0

        lax.fori_loop(0, C, diff, 0)
        pltpu.sync_copy(buf_a, acc.at[idx_r], add=True)
        pltpu.sync_copy(buf_b, acc.at[idx_c], add=True)
        return 0

    lax.fori_loop(0, NCHUNK, chunk, 0)

    plsc.subcore_barrier()

    @pl.when(c == 0)
    def _():
        pltpu.sync_copy(acc.at[pl.ds(r0, RPT)], sa.at[pl.ds(r0, RPT)])

    @pl.when(c == 1)
    def _():
        pltpu.sync_copy(acc.at[pl.ds(r0, RPT)], sb.at[pl.ds(r0, RPT)])


_sc_kernel = functools.partial(
    pl.kernel,
    out_type=[
        jax.ShapeDtypeStruct((NP, D), jnp.float32),
        jax.ShapeDtypeStruct((NP, D), jnp.float32),
    ],
    mesh=plsc.VectorSubcoreMesh(
        core_axis_name="c", subcore_axis_name="s",
        num_cores=NC, num_subcores=NS,
    ),
    scratch_types=[
        pltpu.VMEM((C,), jnp.int32),
        pltpu.VMEM((C,), jnp.int32),
        pltpu.VMEM((C, D), jnp.float32),
        pltpu.VMEM((C, D), jnp.float32),
        pltpu.VMEM_SHARED((NP, D), jnp.float32),
        pltpu.SemaphoreType.DMA,
    ],
)(_sc_body)


@jax.jit
def kernel(z, x, edge_index, norm_factor):
    del x
    zr = _prepass(z, norm_factor)
    rows = edge_index[0]
    cols = edge_index[1]
    zeros = jnp.zeros((NP, D), jnp.float32)
    sa, sb = _sc_kernel(zr, rows, cols, zeros)
    return _postpass(z, norm_factor, sa, sb)
